# baseline (speedup 1.0000x reference)
import jax
import jax.numpy as jnp
from jax import lax
from jax.experimental import pallas as pl
from jax.experimental.pallas import tpu as pltpu
from jax.experimental.pallas import tpu_sc as plsc

_N = 32768
_NV = _N // 16
_NC = 2
_NW = 32
_ROWS_PER_W = 4


def _sc_body(x_hbm, k_hbm, val_hbm, idx_hbm, xrow, hist, kbuf, resv, resi):
    imin = jnp.int32(-2147483648)
    lane = lax.iota(jnp.int32, 16)
    c31 = jnp.full((16,), 31, jnp.int32)
    cimin = jnp.full((16,), imin, jnp.int32)
    ones = jnp.ones((16,), jnp.int32)
    zeros16 = jnp.zeros((16,), jnp.int32)

    wid = lax.axis_index("s") * _NC + lax.axis_index("c")

    pltpu.sync_copy(k_hbm, kbuf)
    kv = kbuf[...]

    def monokey(i):
        xv = xrow[pl.ds(i * 16, 16)]
        b = lax.bitcast_convert_type(xv, jnp.int32)
        asr = lax.shift_right_arithmetic(b, c31)
        return lax.bitwise_xor(b, lax.bitwise_or(asr, cimin))

    def zero_hist(nbins):
        def zb(i, c):
            hist[pl.ds(i * 16, 16)] = zeros16
            return c
        lax.fori_loop(0, nbins // 16, zb, 0)

    def hist_pass(digit_mask_fn):
        def body(i, c):
            d, m = digit_mask_fn(monokey(i))
            plsc.addupdate_scatter(hist, [d], ones, mask=m)
            return c
        lax.fori_loop(0, _NV, body, 0)

    def scan_hist(nbins, kcur):
        def body(i, carry):
            tot_carry, b_acc, cb_acc = carry
            hv = hist[pl.ds(i * 16, 16)]
            cs = plsc.cumsum(hv)
            tot = cs + tot_carry
            excl = tot - hv
            sel = (tot >= kcur) & (excl < kcur)
            gidx = lane + i * 16
            b_acc = b_acc + jnp.where(sel, gidx, 0)
            cb_acc = cb_acc + jnp.where(sel, excl, 0)
            return tot_carry + jnp.max(cs), b_acc, cb_acc
        _, b_acc, cb_acc = lax.fori_loop(
            0, nbins // 16, body, (zeros16, zeros16, zeros16))
        return jnp.max(b_acc), jnp.max(cb_acc)

    val_acc = jnp.zeros((16,), jnp.float32)
    idx_acc = jnp.zeros((16,), jnp.int32)

    for r in range(_ROWS_PER_W):
        row = wid * _ROWS_PER_W + r
        pltpu.sync_copy(x_hbm.at[row], xrow)

        zero_hist(2048)
        hist_pass(lambda u: (lax.shift_right_logical(u, 21), None))
        b1, cb1 = scan_hist(2048, kv)
        k2 = kv - cb1

        zero_hist(2048)

        def dm2(u):
            d1 = lax.shift_right_logical(u, 21)
            d2 = lax.bitwise_and(lax.shift_right_logical(u, 10), 0x7FF)
            return d2, d1 == b1
        hist_pass(dm2)
        b2, cb2 = scan_hist(2048, k2)
        k3 = k2 - cb2

        zero_hist(1024)

        def dm3(u):
            d1 = lax.shift_right_logical(u, 21)
            d2 = lax.bitwise_and(lax.shift_right_logical(u, 10), 0x7FF)
            d3 = lax.bitwise_and(u, 0x3FF)
            return d3, (d1 == b1) & (d2 == b2)
        hist_pass(dm3)
        b3, cb3 = scan_hist(1024, k3)

        ustar = (b1 * 2097152) + (b2 * 1024) + b3
        sstar = lax.bitwise_xor(ustar, imin)
        m0 = k3 - cb3 - 1

        def lbody(i, carry):
            eqcnt, ans = carry
            u = monokey(i)
            s = lax.bitwise_xor(u, cimin)
            meq = s == sstar
            csv = plsc.cumsum(meq.astype(jnp.int32))
            sel = meq & ((csv + eqcnt) == (m0 + 1))
            colv = lane + i * 16
            ans = jnp.maximum(ans, jnp.where(sel, colv, -1))
            eqcnt = eqcnt + plsc.all_reduce_population_count(meq)
            return eqcnt, ans
        _, ans = lax.fori_loop(0, _NV, lbody,
                               (zeros16, jnp.full((16,), -1, jnp.int32)))
        col = jnp.max(ans)

        ustar_v = jnp.broadcast_to(ustar, (16,))
        bits_v = jnp.where(ustar_v < 0, lax.bitwise_xor(ustar_v, cimin),
                           lax.bitwise_not(ustar_v))
        val_v = lax.bitcast_convert_type(bits_v, jnp.float32)
        val_acc = jnp.where(lane == r, val_v, val_acc)
        idx_acc = jnp.where(lane == r, col, idx_acc)

    resv[...] = val_acc
    resi[...] = idx_acc
    pltpu.sync_copy(resv, val_hbm.at[wid])
    pltpu.sync_copy(resi, idx_hbm.at[wid])


def _kth_select_sc(x, k_arr):
    mesh = plsc.VectorSubcoreMesh(core_axis_name="c", subcore_axis_name="s")
    f = pl.kernel(
        _sc_body,
        out_type=[
            jax.ShapeDtypeStruct((_NW, 16), jnp.float32),
            jax.ShapeDtypeStruct((_NW, 16), jnp.int32),
        ],
        mesh=mesh,
        compiler_params=pltpu.CompilerParams(needs_layout_passes=False),
        scratch_types=[
            pltpu.VMEM((_N,), jnp.float32),
            pltpu.VMEM((2048,), jnp.int32),
            pltpu.VMEM((16,), jnp.int32),
            pltpu.VMEM((16,), jnp.float32),
            pltpu.VMEM((16,), jnp.int32),
        ],
    )
    return f(x, k_arr)


def kernel(x, k, dim, keepdim, values, indices):
    k_arr = jnp.full((16,), jnp.asarray(k, jnp.int32))
    vals, idxs = _kth_select_sc(x, k_arr)
    kth_val = vals[:, :_ROWS_PER_W].reshape(128, 1)
    kth_idx = idxs[:, :_ROWS_PER_W].reshape(128, 1)
    zero = (jnp.asarray(dim, jnp.int32) - 1) + (
        jnp.asarray(keepdim).astype(jnp.int32) - 1)
    kth_val = (kth_val + zero.astype(kth_val.dtype)).astype(values.dtype)
    kth_idx = (kth_idx + zero).astype(indices.dtype)
    return kth_val, kth_idx


# speedup vs baseline: 1.9084x; 1.9084x over previous
import jax
import jax.numpy as jnp
from jax import lax
from jax.experimental import pallas as pl
from jax.experimental.pallas import tpu as pltpu
from jax.experimental.pallas import tpu_sc as plsc

_N = 32768
_NV = _N // 16
_NC = 2
_NW = 32
_ROWS_PER_W = 4
_U = 4


def _sc_body(x_hbm, k_hbm, val_hbm, idx_hbm,
             xbuf0, xbuf1, ubuf, hist1, hist2, colsum2, hist3, colsum3,
             kbuf, resv, resi, sem0, sem1):
    imin = jnp.int32(-2147483648)
    lane = lax.iota(jnp.int32, 16)
    c31 = jnp.full((16,), 31, jnp.int32)
    cimin = jnp.full((16,), imin, jnp.int32)
    ones = jnp.ones((16,), jnp.int32)
    zeros16 = jnp.zeros((16,), jnp.int32)

    wid = lax.axis_index("s") * _NC + lax.axis_index("c")
    xbufs = (xbuf0, xbuf1)
    sems = (sem0, sem1)

    copies = [pltpu.async_copy(x_hbm.at[wid * _ROWS_PER_W], xbuf0, sem0)]
    pltpu.sync_copy(k_hbm, kbuf)
    kv = kbuf[...]

    def zinit(i, c):
        hist1[pl.ds(i * 16, 16)] = zeros16
        hist2[pl.ds(i * 16, 16)] = zeros16
        colsum2[pl.ds(i * 16, 16)] = zeros16
        return c
    lax.fori_loop(0, 128, zinit, 0)

    def monokey(xref, i):
        xv = xref[pl.ds(i * 16, 16)]
        b = lax.bitcast_convert_type(xv, jnp.int32)
        asr = lax.shift_right_arithmetic(b, c31)
        return lax.bitwise_xor(b, lax.bitwise_or(asr, cimin))

    def scan_hist(href, csref, nbins, kcur, zero_after):
        def body(i, carry):
            tot_carry, b_acc, cb_acc, cnt_acc, col_acc = carry
            hv = href[pl.ds(i * 16, 16)]
            if csref is not None:
                cv = csref[pl.ds(i * 16, 16)]
            if zero_after:
                href[pl.ds(i * 16, 16)] = zeros16
                if csref is not None:
                    csref[pl.ds(i * 16, 16)] = zeros16
            cs = plsc.cumsum(hv)
            tot = cs + tot_carry
            excl = tot - hv
            sel = (tot >= kcur) & (excl < kcur)
            gidx = lane + i * 16
            b_acc = b_acc + jnp.where(sel, gidx, 0)
            cb_acc = cb_acc + jnp.where(sel, excl, 0)
            cnt_acc = cnt_acc + jnp.where(sel, hv, 0)
            if csref is not None:
                col_acc = col_acc + jnp.where(sel, cv, 0)
            return tot_carry + jnp.max(cs), b_acc, cb_acc, cnt_acc, col_acc
        _, b_acc, cb_acc, cnt_acc, col_acc = lax.fori_loop(
            0, nbins // 16, body, (zeros16,) * 5)
        return (jnp.max(b_acc), jnp.max(cb_acc), jnp.max(cnt_acc),
                jnp.max(col_acc))

    val_acc = jnp.zeros((16,), jnp.float32)
    idx_acc = jnp.zeros((16,), jnp.int32)

    for r in range(_ROWS_PER_W):
        xrow = xbufs[r % 2]
        copies[r].wait()
        if r + 1 < _ROWS_PER_W:
            copies.append(pltpu.async_copy(
                x_hbm.at[wid * _ROWS_PER_W + r + 1],
                xbufs[(r + 1) % 2], sems[(r + 1) % 2]))

        def p1(i, c):
            for j in range(_U):
                u = monokey(xrow, i * _U + j)
                ubuf[pl.ds((i * _U + j) * 16, 16)] = u
                plsc.addupdate_scatter(
                    hist1, [lax.shift_right_logical(u, 21)], ones)
            return c
        lax.fori_loop(0, _NV // _U, p1, 0)
        b1, cb1, cnt1, _ = scan_hist(hist1, None, 2048, kv, True)
        k2 = kv - cb1

        def p2(i, c):
            for j in range(_U):
                ii = i * _U + j
                u = ubuf[pl.ds(ii * 16, 16)]
                d1 = lax.shift_right_logical(u, 21)
                d2 = lax.bitwise_and(lax.shift_right_logical(u, 10), 0x7FF)
                m = d1 == b1
                colv = lane + ii * 16
                plsc.addupdate_scatter(hist2, [d2], ones, mask=m)
                plsc.addupdate_scatter(colsum2, [d2], colv, mask=m)
            return c
        lax.fori_loop(0, _NV // _U, p2, 0)
        b2, cb2, cnt2, col2 = scan_hist(hist2, colsum2, 2048, k2, True)
        k3 = k2 - cb2

        def fast_case(_):
            uv = plsc.load_gather(ubuf, [jnp.broadcast_to(col2, (16,))])
            return uv, col2

        def slow_case(_):
            def z3(i, c):
                hist3[pl.ds(i * 16, 16)] = zeros16
                colsum3[pl.ds(i * 16, 16)] = zeros16
                return c
            lax.fori_loop(0, 64, z3, 0)

            def p3(i, c):
                for j in range(_U):
                    ii = i * _U + j
                    u = ubuf[pl.ds(ii * 16, 16)]
                    d1 = lax.shift_right_logical(u, 21)
                    d2 = lax.bitwise_and(
                        lax.shift_right_logical(u, 10), 0x7FF)
                    d3 = lax.bitwise_and(u, 0x3FF)
                    m = (d1 == b1) & (d2 == b2)
                    colv = lane + ii * 16
                    plsc.addupdate_scatter(hist3, [d3], ones, mask=m)
                    plsc.addupdate_scatter(colsum3, [d3], colv, mask=m)
                return c
            lax.fori_loop(0, _NV // _U, p3, 0)
            b3, cb3, cnt3, col3 = scan_hist(hist3, colsum3, 1024, k3, False)
            ustar = lax.bitwise_or(
                lax.bitwise_or(lax.shift_left(b1, 21), lax.shift_left(b2, 10)),
                b3)
            ustar_v = jnp.broadcast_to(ustar, (16,))

            def tie_case(_):
                m0 = k3 - cb3 - 1

                def lbody(i, carry):
                    eqcnt, ans = carry
                    u = ubuf[pl.ds(i * 16, 16)]
                    meq = u == ustar_v
                    csv = plsc.cumsum(meq.astype(jnp.int32))
                    sel = meq & ((csv + eqcnt) == (m0 + 1))
                    colv = lane + i * 16
                    ans = jnp.maximum(ans, jnp.where(sel, colv, -1))
                    eqcnt = eqcnt + plsc.all_reduce_population_count(meq)
                    return eqcnt, ans
                _, ans = lax.fori_loop(
                    0, _NV, lbody,
                    (zeros16, jnp.full((16,), -1, jnp.int32)))
                return jnp.max(ans)

            col = lax.cond(cnt3 == 1, lambda _: col3, tie_case, 0)
            return ustar_v, col

        uv, col = lax.cond(cnt2 == 1, fast_case, slow_case, 0)
        bits_v = jnp.where(uv < 0, lax.bitwise_xor(uv, cimin),
                           lax.bitwise_not(uv))
        val_v = lax.bitcast_convert_type(bits_v, jnp.float32)
        val_acc = jnp.where(lane == r, val_v, val_acc)
        idx_acc = jnp.where(lane == r, col, idx_acc)

    resv[...] = val_acc
    resi[...] = idx_acc
    pltpu.sync_copy(resv, val_hbm.at[wid])
    pltpu.sync_copy(resi, idx_hbm.at[wid])


def _kth_select_sc(x, k_arr):
    mesh = plsc.VectorSubcoreMesh(core_axis_name="c", subcore_axis_name="s")
    f = pl.kernel(
        _sc_body,
        out_type=[
            jax.ShapeDtypeStruct((_NW, 16), jnp.float32),
            jax.ShapeDtypeStruct((_NW, 16), jnp.int32),
        ],
        mesh=mesh,
        compiler_params=pltpu.CompilerParams(needs_layout_passes=False),
        scratch_types=[
            pltpu.VMEM((_N,), jnp.float32),
            pltpu.VMEM((_N,), jnp.float32),
            pltpu.VMEM((_N,), jnp.int32),
            pltpu.VMEM((2048,), jnp.int32),
            pltpu.VMEM((2048,), jnp.int32),
            pltpu.VMEM((2048,), jnp.int32),
            pltpu.VMEM((1024,), jnp.int32),
            pltpu.VMEM((1024,), jnp.int32),
            pltpu.VMEM((16,), jnp.int32),
            pltpu.VMEM((16,), jnp.float32),
            pltpu.VMEM((16,), jnp.int32),
            pltpu.SemaphoreType.DMA,
            pltpu.SemaphoreType.DMA,
        ],
    )
    return f(x, k_arr)


def kernel(x, k, dim, keepdim, values, indices):
    k_arr = jnp.full((16,), jnp.asarray(k, jnp.int32))
    vals, idxs = _kth_select_sc(x, k_arr)
    kth_val = vals[:, :_ROWS_PER_W].reshape(128, 1)
    kth_idx = idxs[:, :_ROWS_PER_W].reshape(128, 1)
    zero = (jnp.asarray(dim, jnp.int32) - 1) + (
        jnp.asarray(keepdim).astype(jnp.int32) - 1)
    kth_val = (kth_val + zero.astype(kth_val.dtype)).astype(values.dtype)
    kth_idx = (kth_idx + zero).astype(indices.dtype)
    return kth_val, kth_idx
